# Initial kernel scaffold; baseline (speedup 1.0000x reference)
import jax
import jax.numpy as jnp
from jax.experimental import pallas as pl


def kernel(x, norm_g, norm_b, in_proj_w, conv_w, conv_b, x_proj_w, dt_proj_w, dt_bias, A_log, D_vec, out_proj_w):
    raise NotImplementedError("write your pallas kernel here")


# speedup vs baseline: 19.6002x; 19.6002x over previous
import functools

import jax
import jax.numpy as jnp
from jax.experimental import pallas as pl
from jax.experimental.pallas import tpu as pltpu

_DIM = 128
_D_STATE = 8
_D_CONV = 4
_D_INNER = 256
_DT_RANK = 8
_EPS = 1e-5


def _body(x_ref, wi_ref, wx_ref, wd_ref, dtb_ref, cw_ref, cb_ref, g_ref,
          bt_ref, dv_ref, alog_ref, wo_ref, o_ref,
          tail_ref, h_ref, dl_ref, du_ref, bb_ref, cc_ref, yy_ref):
    l = pl.program_id(1)
    T, NBB, C = x_ref.shape
    DI = _D_INNER

    @pl.when(l == 0)
    def _init():
        tail_ref[...] = jnp.zeros_like(tail_ref)
        h_ref[...] = jnp.zeros_like(h_ref)

    xb = x_ref[...]
    mu = jnp.mean(xb, axis=-1, keepdims=True)
    xc = xb - mu
    var = jnp.mean(xc * xc, axis=-1, keepdims=True)
    xn = xc * jax.lax.rsqrt(var + _EPS) * g_ref[...] + bt_ref[...]

    xz = jnp.dot(xn.reshape(T * NBB, C), wi_ref[...],
                 preferred_element_type=jnp.float32)
    xp = xz[:, :DI].reshape(T, NBB, DI)
    zg = xz[:, DI:].reshape(T, NBB, DI)

    tail_prev = tail_ref[...]
    xpad = jnp.concatenate([tail_prev, xp], axis=0)
    tail_ref[...] = xp[T - (_D_CONV - 1):, :, :]
    conv = jnp.broadcast_to(cb_ref[...], (T, NBB, DI))
    for k in range(_D_CONV):
        conv = conv + xpad[k:k + T] * cw_ref[k:k + 1, :].reshape(1, 1, DI)
    u = conv * jax.nn.sigmoid(conv)

    x2 = jnp.dot(u.reshape(T * NBB, DI), wx_ref[...],
                 preferred_element_type=jnp.float32)
    dt2 = x2[:, :_DT_RANK]
    da = jnp.dot(dt2, wd_ref[...], preferred_element_type=jnp.float32) \
        + dtb_ref[...]
    delta = jnp.maximum(da, 0.0) + jnp.log(1.0 + jnp.exp(-jnp.abs(da)))
    dl3 = delta.reshape(T, NBB, DI)
    dl_ref[...] = dl3
    du_ref[...] = dl3 * u
    bb_ref[...] = x2[:, _DT_RANK:_DT_RANK + _D_STATE].reshape(T, NBB, _D_STATE)
    cc_ref[...] = x2[:, _DT_RANK + _D_STATE:].reshape(T, NBB, _D_STATE)

    at = -jnp.exp(alog_ref[...])
    a_rows = [at[s:s + 1, :] for s in range(_D_STATE)]
    hs0 = tuple(h_ref[s] for s in range(_D_STATE))

    def step(t, hs):
        d_t = dl_ref[t]
        du_t = du_ref[t]
        brow = bb_ref[t]
        crow = cc_ref[t]
        acc = jnp.zeros((NBB, DI), jnp.float32)
        new = []
        for s in range(_D_STATE):
            h = jnp.exp(d_t * a_rows[s]) * hs[s] + du_t * brow[:, s:s + 1]
            acc = acc + h * crow[:, s:s + 1]
            new.append(h)
        yy_ref[t] = acc
        return tuple(new)

    hsf = jax.lax.fori_loop(0, T, step, hs0)
    for s in range(_D_STATE):
        h_ref[s] = hsf[s]

    y = yy_ref[...] + u * dv_ref[...]
    y = y * (zg * jax.nn.sigmoid(zg))
    out = jnp.dot(y.reshape(T * NBB, DI), wo_ref[...],
                  preferred_element_type=jnp.float32)
    o_ref[...] = xb + out.reshape(T, NBB, C)


@functools.partial(jax.jit, static_argnames=())
def kernel(x, norm_g, norm_b, in_proj_w, conv_w, conv_b, x_proj_w, dt_proj_w,
           dt_bias, A_log, D_vec, out_proj_w):
    B, C, Z, H, W = x.shape
    p1, p2, p3 = 2, 2, 2
    NZ, NH, NW = Z // p1, H // p2, W // p3
    NB, L = B * p1 * p2 * p3, NZ * NH * NW
    DI = _D_INNER

    xd = x.reshape(B, C, NZ, p1, NH, p2, NW, p3).transpose(2, 4, 6, 0, 3, 5, 7, 1)
    xf = xd.reshape(L, NB, C)

    T = L
    for cand in (200, 160, 80, 40, 16, 8):
        if L % cand == 0:
            T = cand
            break
    NC = L // T
    NBB = NB // 2

    f32 = jnp.float32
    yo = pl.pallas_call(
        _body,
        grid=(2, NC),
        in_specs=[
            pl.BlockSpec((T, NBB, C), lambda c, l: (l, c, 0)),
            pl.BlockSpec((C, 2 * DI), lambda c, l: (0, 0)),
            pl.BlockSpec((DI, _DT_RANK + 2 * _D_STATE), lambda c, l: (0, 0)),
            pl.BlockSpec((_DT_RANK, DI), lambda c, l: (0, 0)),
            pl.BlockSpec((1, DI), lambda c, l: (0, 0)),
            pl.BlockSpec((_D_CONV, DI), lambda c, l: (0, 0)),
            pl.BlockSpec((1, 1, DI), lambda c, l: (0, 0, 0)),
            pl.BlockSpec((1, 1, C), lambda c, l: (0, 0, 0)),
            pl.BlockSpec((1, 1, C), lambda c, l: (0, 0, 0)),
            pl.BlockSpec((1, 1, DI), lambda c, l: (0, 0, 0)),
            pl.BlockSpec((_D_STATE, DI), lambda c, l: (0, 0)),
            pl.BlockSpec((DI, C), lambda c, l: (0, 0)),
        ],
        out_specs=pl.BlockSpec((T, NBB, C), lambda c, l: (l, c, 0)),
        out_shape=jax.ShapeDtypeStruct((L, NB, C), f32),
        scratch_shapes=[
            pltpu.VMEM((_D_CONV - 1, NBB, DI), f32),
            pltpu.VMEM((_D_STATE, NBB, DI), f32),
            pltpu.VMEM((T, NBB, DI), f32),
            pltpu.VMEM((T, NBB, DI), f32),
            pltpu.VMEM((T, NBB, _D_STATE), f32),
            pltpu.VMEM((T, NBB, _D_STATE), f32),
            pltpu.VMEM((T, NBB, DI), f32),
        ],
        compiler_params=pltpu.CompilerParams(
            dimension_semantics=("parallel", "arbitrary"),
            vmem_limit_bytes=48 * 1024 * 1024,
        ),
    )(
        xf,
        in_proj_w.T, x_proj_w.T, dt_proj_w.T, dt_bias.reshape(1, DI),
        conv_w.T, conv_b.reshape(1, 1, DI),
        norm_g.reshape(1, 1, C), norm_b.reshape(1, 1, C),
        D_vec.reshape(1, 1, DI), A_log.T, out_proj_w.T,
    )

    out = yo.reshape(NZ, NH, NW, B, p1, p2, p3, C)
    out = out.transpose(3, 7, 0, 4, 1, 5, 2, 6).reshape(B, C, Z, H, W)
    return out


# speedup vs baseline: 36.5532x; 1.8649x over previous
import functools

import jax
import jax.numpy as jnp
from jax.experimental import pallas as pl
from jax.experimental.pallas import tpu as pltpu

_DIM = 128
_D_STATE = 8
_D_CONV = 4
_D_INNER = 256
_DT_RANK = 8
_EPS = 1e-5


def _body(x_ref, wi_ref, wx_ref, wd_ref, dtb_ref, cw_ref, cb_ref, g_ref,
          bt_ref, dv_ref, alog_ref, wo_ref, o_ref,
          tail_ref, h_ref, ea_ref, dub_ref, cb3_ref, yy_ref):
    l = pl.program_id(1)
    T, NBB, C = x_ref.shape
    DI = _D_INNER

    @pl.when(l == 0)
    def _init():
        tail_ref[...] = jnp.zeros_like(tail_ref)
        h_ref[...] = jnp.zeros_like(h_ref)

    xb = x_ref[...]
    mu = jnp.mean(xb, axis=-1, keepdims=True)
    xc = xb - mu
    var = jnp.mean(xc * xc, axis=-1, keepdims=True)
    xn = xc * jax.lax.rsqrt(var + _EPS) * g_ref[...] + bt_ref[...]

    xz = jnp.dot(xn.reshape(T * NBB, C), wi_ref[...],
                 preferred_element_type=jnp.float32)
    xp = xz[:, :DI].reshape(T, NBB, DI)
    zg = xz[:, DI:].reshape(T, NBB, DI)

    tail_prev = tail_ref[...]
    xpad = jnp.concatenate([tail_prev, xp], axis=0)
    tail_ref[...] = xp[T - (_D_CONV - 1):, :, :]
    conv = jnp.broadcast_to(cb_ref[...], (T, NBB, DI))
    for k in range(_D_CONV):
        conv = conv + xpad[k:k + T] * cw_ref[k:k + 1, :].reshape(1, 1, DI)
    u = conv * jax.nn.sigmoid(conv)

    x2 = jnp.dot(u.reshape(T * NBB, DI), wx_ref[...],
                 preferred_element_type=jnp.float32)
    dt2 = x2[:, :_DT_RANK]
    da = jnp.dot(dt2, wd_ref[...], preferred_element_type=jnp.float32) \
        + dtb_ref[...]
    delta = jnp.maximum(da, 0.0) + jnp.log(1.0 + jnp.exp(-jnp.abs(da)))
    dl3 = delta.reshape(T, NBB, DI)
    du3 = dl3 * u
    bb3 = x2[:, _DT_RANK:_DT_RANK + _D_STATE].reshape(T, NBB, _D_STATE)
    cc3 = x2[:, _DT_RANK + _D_STATE:].reshape(T, NBB, _D_STATE)

    at = -jnp.exp(alog_ref[...])
    for s in range(_D_STATE):
        ea_ref[s] = jnp.exp(dl3 * at[s:s + 1, :].reshape(1, 1, DI))
        dub_ref[s] = du3 * bb3[:, :, s:s + 1]
        cb3_ref[s] = jnp.broadcast_to(cc3[:, :, s:s + 1], (T, NBB, DI))

    hs0 = tuple(h_ref[s] for s in range(_D_STATE))

    def step(t, hs):
        acc = None
        new = []
        for s in range(_D_STATE):
            h = ea_ref[s, t] * hs[s] + dub_ref[s, t]
            c = h * cb3_ref[s, t]
            acc = c if acc is None else acc + c
            new.append(h)
        yy_ref[t] = acc
        return tuple(new)

    UNROLL = 4

    def step4(i, hs):
        t0 = i * UNROLL
        for j in range(UNROLL):
            hs = step(t0 + j, hs)
        return hs

    hsf = jax.lax.fori_loop(0, T // UNROLL, step4, hs0)
    for s in range(_D_STATE):
        h_ref[s] = hsf[s]

    y = yy_ref[...] + u * dv_ref[...]
    y = y * (zg * jax.nn.sigmoid(zg))
    out = jnp.dot(y.reshape(T * NBB, DI), wo_ref[...],
                  preferred_element_type=jnp.float32)
    o_ref[...] = xb + out.reshape(T, NBB, C)


@functools.partial(jax.jit, static_argnames=())
def kernel(x, norm_g, norm_b, in_proj_w, conv_w, conv_b, x_proj_w, dt_proj_w,
           dt_bias, A_log, D_vec, out_proj_w):
    B, C, Z, H, W = x.shape
    p1, p2, p3 = 2, 2, 2
    NZ, NH, NW = Z // p1, H // p2, W // p3
    NB, L = B * p1 * p2 * p3, NZ * NH * NW
    DI = _D_INNER

    xd = x.reshape(B, C, NZ, p1, NH, p2, NW, p3).transpose(2, 4, 6, 0, 3, 5, 7, 1)
    xf = xd.reshape(L, NB, C)

    T = L
    for cand in (80, 40, 16, 8):
        if L % cand == 0:
            T = cand
            break
    NC = L // T
    NBB = NB // 2

    f32 = jnp.float32
    yo = pl.pallas_call(
        _body,
        grid=(2, NC),
        in_specs=[
            pl.BlockSpec((T, NBB, C), lambda c, l: (l, c, 0)),
            pl.BlockSpec((C, 2 * DI), lambda c, l: (0, 0)),
            pl.BlockSpec((DI, _DT_RANK + 2 * _D_STATE), lambda c, l: (0, 0)),
            pl.BlockSpec((_DT_RANK, DI), lambda c, l: (0, 0)),
            pl.BlockSpec((1, DI), lambda c, l: (0, 0)),
            pl.BlockSpec((_D_CONV, DI), lambda c, l: (0, 0)),
            pl.BlockSpec((1, 1, DI), lambda c, l: (0, 0, 0)),
            pl.BlockSpec((1, 1, C), lambda c, l: (0, 0, 0)),
            pl.BlockSpec((1, 1, C), lambda c, l: (0, 0, 0)),
            pl.BlockSpec((1, 1, DI), lambda c, l: (0, 0, 0)),
            pl.BlockSpec((_D_STATE, DI), lambda c, l: (0, 0)),
            pl.BlockSpec((DI, C), lambda c, l: (0, 0)),
        ],
        out_specs=pl.BlockSpec((T, NBB, C), lambda c, l: (l, c, 0)),
        out_shape=jax.ShapeDtypeStruct((L, NB, C), f32),
        scratch_shapes=[
            pltpu.VMEM((_D_CONV - 1, NBB, DI), f32),
            pltpu.VMEM((_D_STATE, NBB, DI), f32),
            pltpu.VMEM((_D_STATE, T, NBB, DI), f32),
            pltpu.VMEM((_D_STATE, T, NBB, DI), f32),
            pltpu.VMEM((_D_STATE, T, NBB, DI), f32),
            pltpu.VMEM((T, NBB, DI), f32),
        ],
        compiler_params=pltpu.CompilerParams(
            dimension_semantics=("parallel", "arbitrary"),
            vmem_limit_bytes=48 * 1024 * 1024,
        ),
    )(
        xf,
        in_proj_w.T, x_proj_w.T, dt_proj_w.T, dt_bias.reshape(1, DI),
        conv_w.T, conv_b.reshape(1, 1, DI),
        norm_g.reshape(1, 1, C), norm_b.reshape(1, 1, C),
        D_vec.reshape(1, 1, DI), A_log.T, out_proj_w.T,
    )

    out = yo.reshape(NZ, NH, NW, B, p1, p2, p3, C)
    out = out.transpose(3, 7, 0, 4, 1, 5, 2, 6).reshape(B, C, Z, H, W)
    return out
